# initial kernel scaffold (unmeasured)
import jax
import jax.numpy as jnp
from jax import lax
from jax.experimental import pallas as pl
from jax.experimental.pallas import tpu as pltpu

N_DEV = 4
HQ_LOC = 8
DH = 128
SQ = 2048
SKV_LOC = 2048
SKV = N_DEV * SKV_LOC
SCALE = 0.08838834764831843
QB = 256


def _all_to_all_kv(Kt, Vt):

    def body(kt_ref, vt_ref, kg_ref, vg_ref,
             send_k, recv_k, send_v, recv_v, loc_sems):
        my = lax.axis_index("i")

        kloc = pltpu.make_async_copy(
            kt_ref.at[my], kg_ref.at[pl.ds(my * SKV_LOC, SKV_LOC)], loc_sems.at[0]
        )
        kloc.start()
        vloc = pltpu.make_async_copy(
            vt_ref.at[my], vg_ref.at[pl.ds(my * SKV_LOC, SKV_LOC)], loc_sems.at[1]
        )
        vloc.start()

        rdmas = []
        for d in range(1, N_DEV):
            t = lax.rem(my + d, N_DEV)
            rk = pltpu.make_async_remote_copy(
                src_ref=kt_ref.at[t],
                dst_ref=kg_ref.at[pl.ds(my * SKV_LOC, SKV_LOC)],
                send_sem=send_k.at[d - 1],
                recv_sem=recv_k.at[d - 1],
                device_id=(t,),
                device_id_type=pl.DeviceIdType.MESH,
            )
            rk.start()
            rv = pltpu.make_async_remote_copy(
                src_ref=vt_ref.at[t],
                dst_ref=vg_ref.at[pl.ds(my * SKV_LOC, SKV_LOC)],
                send_sem=send_v.at[d - 1],
                recv_sem=recv_v.at[d - 1],
                device_id=(t,),
                device_id_type=pl.DeviceIdType.MESH,
            )
            rv.start()
            rdmas.append((rk, rv))

        kloc.wait()
        vloc.wait()
        for rk, rv in rdmas:
            rk.wait()
            rv.wait()

    return pl.pallas_call(
        body,
        out_shape=[
            jax.ShapeDtypeStruct((SKV, HQ_LOC, DH), Kt.dtype),
            jax.ShapeDtypeStruct((SKV, HQ_LOC, DH), Vt.dtype),
        ],
        in_specs=[
            pl.BlockSpec(memory_space=pltpu.ANY),
            pl.BlockSpec(memory_space=pltpu.ANY),
        ],
        out_specs=[
            pl.BlockSpec(memory_space=pltpu.ANY),
            pl.BlockSpec(memory_space=pltpu.ANY),
        ],
        scratch_shapes=[
            pltpu.SemaphoreType.DMA((N_DEV - 1,)),
            pltpu.SemaphoreType.DMA((N_DEV - 1,)),
            pltpu.SemaphoreType.DMA((N_DEV - 1,)),
            pltpu.SemaphoreType.DMA((N_DEV - 1,)),
            pltpu.SemaphoreType.DMA((2,)),
        ],
        compiler_params=pltpu.CompilerParams(collective_id=0),
    )(Kt, Vt)


def _attention(Q, Kg, Vg):

    def body(q_ref, k_ref, v_ref, o_ref):
        qb = pl.program_id(1)
        q = q_ref[:, 0, :]
        k = k_ref[:, 0, :]
        v = v_ref[:, 0, :]
        s = jax.lax.dot_general(
            q, k, (((1,), (1,)), ((), ())),
            preferred_element_type=jnp.float32,
        ) * SCALE
        qi = qb * QB + lax.broadcasted_iota(jnp.int32, (QB, SKV), 0)
        ki = lax.broadcasted_iota(jnp.int32, (QB, SKV), 1)
        mask = (jnp.abs(qi - ki) <= 128) | (ki < 32) | (qi < 32)
        s = jnp.where(mask, s, -1e9)
        m = jnp.max(s, axis=-1, keepdims=True)
        w = jnp.exp(s - m)
        w = w / jnp.sum(w, axis=-1, keepdims=True)
        o_ref[:, 0, :] = jax.lax.dot_general(
            w, v, (((1,), (0,)), ((), ())),
            preferred_element_type=jnp.float32,
        )

    return pl.pallas_call(
        body,
        grid=(HQ_LOC, SQ // QB),
        in_specs=[
            pl.BlockSpec((QB, 1, DH), lambda h, qb: (qb, h, 0)),
            pl.BlockSpec((SKV, 1, DH), lambda h, qb: (0, h, 0)),
            pl.BlockSpec((SKV, 1, DH), lambda h, qb: (0, h, 0)),
        ],
        out_specs=pl.BlockSpec((QB, 1, DH), lambda h, qb: (qb, h, 0)),
        out_shape=jax.ShapeDtypeStruct((SQ, HQ_LOC, DH), jnp.float32),
    )(Q, Kg, Vg)


def _all_gather(partial):

    def body(p_ref, g_ref, send, recv, loc_sem):
        my = lax.axis_index("i")
        loc = pltpu.make_async_copy(p_ref, g_ref.at[my], loc_sem)
        loc.start()
        rdmas = []
        for d in range(1, N_DEV):
            t = lax.rem(my + d, N_DEV)
            r = pltpu.make_async_remote_copy(
                src_ref=p_ref,
                dst_ref=g_ref.at[my],
                send_sem=send.at[d - 1],
                recv_sem=recv.at[d - 1],
                device_id=(t,),
                device_id_type=pl.DeviceIdType.MESH,
            )
            r.start()
            rdmas.append(r)
        loc.wait()
        for r in rdmas:
            r.wait()

    return pl.pallas_call(
        body,
        out_shape=jax.ShapeDtypeStruct((N_DEV, SQ, HQ_LOC * DH), partial.dtype),
        in_specs=[pl.BlockSpec(memory_space=pltpu.ANY)],
        out_specs=pl.BlockSpec(memory_space=pltpu.ANY),
        scratch_shapes=[
            pltpu.SemaphoreType.DMA((N_DEV - 1,)),
            pltpu.SemaphoreType.DMA((N_DEV - 1,)),
            pltpu.SemaphoreType.DMA(),
        ],
        compiler_params=pltpu.CompilerParams(collective_id=1),
    )(partial)


def kernel(x, Wq, K_ext, V_ext, Wo):
    Q = (x[0] @ Wq).reshape(SQ, HQ_LOC, DH)

    Kt = K_ext[0].reshape(SKV_LOC, N_DEV, HQ_LOC, DH).transpose(1, 0, 2, 3)
    Vt = V_ext[0].reshape(SKV_LOC, N_DEV, HQ_LOC, DH).transpose(1, 0, 2, 3)

    Kg, Vg = _all_to_all_kv(Kt, Vt)
    ctx = _attention(Q, Kg, Vg)

    partial = ctx.reshape(SQ, HQ_LOC * DH) @ Wo
    gathered = _all_gather(partial)
    return jnp.sum(gathered, axis=0)[None]


# baseline (device time: 1227834 ns/iter reference)
import jax
import jax.numpy as jnp
from jax import lax
from jax.experimental import pallas as pl
from jax.experimental.pallas import tpu as pltpu

N_DEV = 4
HQ_LOC = 8
DH = 128
SQ = 2048
SKV_LOC = 2048
SKV = N_DEV * SKV_LOC
SCALE = 0.08838834764831843
QB = 256


def _all_to_all_kv(Kt, Vt):

    def body(kt_ref, vt_ref, kg_ref, vg_ref,
             send_k, recv_k, send_v, recv_v, loc_sems):
        my = lax.axis_index("i")

        barrier = pltpu.get_barrier_semaphore()
        for d in range(1, N_DEV):
            pl.semaphore_signal(
                barrier, inc=1,
                device_id=(lax.rem(my + d, N_DEV),),
                device_id_type=pl.DeviceIdType.MESH,
            )
        pl.semaphore_wait(barrier, N_DEV - 1)

        kloc = pltpu.make_async_copy(
            kt_ref.at[my], kg_ref.at[:, pl.ds(my * SKV_LOC, SKV_LOC), :], loc_sems.at[0]
        )
        kloc.start()
        vloc = pltpu.make_async_copy(
            vt_ref.at[my], vg_ref.at[:, pl.ds(my * SKV_LOC, SKV_LOC), :], loc_sems.at[1]
        )
        vloc.start()

        rdmas = []
        for d in range(1, N_DEV):
            t = lax.rem(my + d, N_DEV)
            rk = pltpu.make_async_remote_copy(
                src_ref=kt_ref.at[t],
                dst_ref=kg_ref.at[:, pl.ds(my * SKV_LOC, SKV_LOC), :],
                send_sem=send_k.at[d - 1],
                recv_sem=recv_k.at[d - 1],
                device_id=(t,),
                device_id_type=pl.DeviceIdType.MESH,
            )
            rk.start()
            rv = pltpu.make_async_remote_copy(
                src_ref=vt_ref.at[t],
                dst_ref=vg_ref.at[:, pl.ds(my * SKV_LOC, SKV_LOC), :],
                send_sem=send_v.at[d - 1],
                recv_sem=recv_v.at[d - 1],
                device_id=(t,),
                device_id_type=pl.DeviceIdType.MESH,
            )
            rv.start()
            rdmas.append((rk, rv))

        kloc.wait()
        vloc.wait()
        for rk, rv in rdmas:
            rk.wait()
            rv.wait()

    return pl.pallas_call(
        body,
        out_shape=[
            jax.ShapeDtypeStruct((HQ_LOC, SKV, DH), Kt.dtype),
            jax.ShapeDtypeStruct((HQ_LOC, SKV, DH), Vt.dtype),
        ],
        in_specs=[
            pl.BlockSpec(memory_space=pl.ANY),
            pl.BlockSpec(memory_space=pl.ANY),
        ],
        out_specs=[
            pl.BlockSpec(memory_space=pl.ANY),
            pl.BlockSpec(memory_space=pl.ANY),
        ],
        scratch_shapes=[
            pltpu.SemaphoreType.DMA((N_DEV - 1,)),
            pltpu.SemaphoreType.DMA((N_DEV - 1,)),
            pltpu.SemaphoreType.DMA((N_DEV - 1,)),
            pltpu.SemaphoreType.DMA((N_DEV - 1,)),
            pltpu.SemaphoreType.DMA((2,)),
        ],
        compiler_params=pltpu.CompilerParams(collective_id=0),
    )(Kt, Vt)


def _attention(Q, Kg, Vg):

    def body(q_ref, k_ref, v_ref, o_ref):
        qb = pl.program_id(1)
        q = q_ref[0]
        k = k_ref[0]
        v = v_ref[0]
        s = jax.lax.dot_general(
            q, k, (((1,), (1,)), ((), ())),
            preferred_element_type=jnp.float32,
        ) * SCALE
        qi = qb * QB + lax.broadcasted_iota(jnp.int32, (QB, SKV), 0)
        ki = lax.broadcasted_iota(jnp.int32, (QB, SKV), 1)
        mask = (jnp.abs(qi - ki) <= 128) | (ki < 32) | (qi < 32)
        s = jnp.where(mask, s, -1e9)
        m = jnp.max(s, axis=-1, keepdims=True)
        w = jnp.exp(s - m)
        w = w / jnp.sum(w, axis=-1, keepdims=True)
        o_ref[0] = jax.lax.dot_general(
            w, v, (((1,), (0,)), ((), ())),
            preferred_element_type=jnp.float32,
        )

    return pl.pallas_call(
        body,
        grid=(HQ_LOC, SQ // QB),
        in_specs=[
            pl.BlockSpec((1, QB, DH), lambda h, qb: (h, qb, 0)),
            pl.BlockSpec((1, SKV, DH), lambda h, qb: (h, 0, 0)),
            pl.BlockSpec((1, SKV, DH), lambda h, qb: (h, 0, 0)),
        ],
        out_specs=pl.BlockSpec((1, QB, DH), lambda h, qb: (h, qb, 0)),
        out_shape=jax.ShapeDtypeStruct((HQ_LOC, SQ, DH), jnp.float32),
    )(Q, Kg, Vg)


def _all_gather(partial):

    def body(p_ref, g_ref, send, recv, loc_sem):
        my = lax.axis_index("i")
        barrier = pltpu.get_barrier_semaphore()
        for d in range(1, N_DEV):
            pl.semaphore_signal(
                barrier, inc=1,
                device_id=(lax.rem(my + d, N_DEV),),
                device_id_type=pl.DeviceIdType.MESH,
            )
        pl.semaphore_wait(barrier, N_DEV - 1)
        loc = pltpu.make_async_copy(p_ref, g_ref.at[my], loc_sem)
        loc.start()
        rdmas = []
        for d in range(1, N_DEV):
            t = lax.rem(my + d, N_DEV)
            r = pltpu.make_async_remote_copy(
                src_ref=p_ref,
                dst_ref=g_ref.at[my],
                send_sem=send.at[d - 1],
                recv_sem=recv.at[d - 1],
                device_id=(t,),
                device_id_type=pl.DeviceIdType.MESH,
            )
            r.start()
            rdmas.append(r)
        loc.wait()
        for r in rdmas:
            r.wait()

    return pl.pallas_call(
        body,
        out_shape=jax.ShapeDtypeStruct((N_DEV, SQ, HQ_LOC * DH), partial.dtype),
        in_specs=[pl.BlockSpec(memory_space=pl.ANY)],
        out_specs=pl.BlockSpec(memory_space=pl.ANY),
        scratch_shapes=[
            pltpu.SemaphoreType.DMA((N_DEV - 1,)),
            pltpu.SemaphoreType.DMA((N_DEV - 1,)),
            pltpu.SemaphoreType.DMA(()),
        ],
        compiler_params=pltpu.CompilerParams(collective_id=1),
    )(partial)


def kernel(x, Wq, K_ext, V_ext, Wo):
    Q = (x[0] @ Wq).reshape(SQ, HQ_LOC, DH).transpose(1, 0, 2)

    Kt = K_ext[0].reshape(SKV_LOC, N_DEV, HQ_LOC, DH).transpose(1, 2, 0, 3)
    Vt = V_ext[0].reshape(SKV_LOC, N_DEV, HQ_LOC, DH).transpose(1, 2, 0, 3)

    Kg, Vg = _all_to_all_kv(Kt, Vt)
    ctx = _attention(Q, Kg, Vg)

    partial = ctx.transpose(1, 0, 2).reshape(SQ, HQ_LOC * DH) @ Wo
    gathered = _all_gather(partial)
    return jnp.sum(gathered, axis=0)[None]


# device time: 591778 ns/iter; 2.0748x vs baseline; 2.0748x over previous
import jax
import jax.numpy as jnp
from jax import lax
from jax.experimental import pallas as pl
from jax.experimental.pallas import tpu as pltpu

N_DEV = 4
HQ_LOC = 8
DH = 128
SQ = 2048
SKV_LOC = 2048
SKV = N_DEV * SKV_LOC
SCALE = 0.08838834764831843
QB = 256


def _all_to_all_kv(Kt, Vt):

    def body(kt_ref, vt_ref, kg_ref, vg_ref,
             send_k, recv_k, send_v, recv_v, loc_sems):
        my = lax.axis_index("i")

        barrier = pltpu.get_barrier_semaphore()
        for d in range(1, N_DEV):
            pl.semaphore_signal(
                barrier, inc=1,
                device_id=(lax.rem(my + d, N_DEV),),
                device_id_type=pl.DeviceIdType.MESH,
            )
        pl.semaphore_wait(barrier, N_DEV - 1)

        kloc = pltpu.make_async_copy(
            kt_ref.at[my], kg_ref.at[:, pl.ds(my * SKV_LOC, SKV_LOC), :], loc_sems.at[0]
        )
        kloc.start()
        vloc = pltpu.make_async_copy(
            vt_ref.at[my], vg_ref.at[:, pl.ds(my * SKV_LOC, SKV_LOC), :], loc_sems.at[1]
        )
        vloc.start()

        rdmas = []
        for d in range(1, N_DEV):
            t = lax.rem(my + d, N_DEV)
            rk = pltpu.make_async_remote_copy(
                src_ref=kt_ref.at[t],
                dst_ref=kg_ref.at[:, pl.ds(my * SKV_LOC, SKV_LOC), :],
                send_sem=send_k.at[d - 1],
                recv_sem=recv_k.at[d - 1],
                device_id=(t,),
                device_id_type=pl.DeviceIdType.MESH,
            )
            rk.start()
            rv = pltpu.make_async_remote_copy(
                src_ref=vt_ref.at[t],
                dst_ref=vg_ref.at[:, pl.ds(my * SKV_LOC, SKV_LOC), :],
                send_sem=send_v.at[d - 1],
                recv_sem=recv_v.at[d - 1],
                device_id=(t,),
                device_id_type=pl.DeviceIdType.MESH,
            )
            rv.start()
            rdmas.append((rk, rv))

        kloc.wait()
        vloc.wait()
        for rk, rv in rdmas:
            rk.wait()
            rv.wait()

    return pl.pallas_call(
        body,
        out_shape=[
            jax.ShapeDtypeStruct((HQ_LOC, SKV, DH), Kt.dtype),
            jax.ShapeDtypeStruct((HQ_LOC, SKV, DH), Vt.dtype),
        ],
        in_specs=[
            pl.BlockSpec(memory_space=pl.ANY),
            pl.BlockSpec(memory_space=pl.ANY),
        ],
        out_specs=[
            pl.BlockSpec(memory_space=pl.ANY),
            pl.BlockSpec(memory_space=pl.ANY),
        ],
        scratch_shapes=[
            pltpu.SemaphoreType.DMA((N_DEV - 1,)),
            pltpu.SemaphoreType.DMA((N_DEV - 1,)),
            pltpu.SemaphoreType.DMA((N_DEV - 1,)),
            pltpu.SemaphoreType.DMA((N_DEV - 1,)),
            pltpu.SemaphoreType.DMA((2,)),
        ],
        compiler_params=pltpu.CompilerParams(collective_id=0),
    )(Kt, Vt)


def _attention_global_rows(Q, Kg, Vg):

    def body(q_ref, k_ref, v_ref, o_ref):
        q = q_ref[0, :32, :]
        k = k_ref[0]
        v = v_ref[0]
        s = jax.lax.dot_general(
            q, k, (((1,), (1,)), ((), ())),
            preferred_element_type=jnp.float32,
        ) * SCALE
        m = jnp.max(s, axis=-1, keepdims=True)
        w = jnp.exp(s - m)
        w = w / jnp.sum(w, axis=-1, keepdims=True)
        o_ref[0] = jax.lax.dot_general(
            w.astype(v.dtype), v, (((1,), (0,)), ((), ())),
            preferred_element_type=jnp.float32,
        )

    return pl.pallas_call(
        body,
        grid=(HQ_LOC,),
        in_specs=[
            pl.BlockSpec((1, QB, DH), lambda h: (h, 0, 0)),
            pl.BlockSpec((1, SKV, DH), lambda h: (h, 0, 0)),
            pl.BlockSpec((1, SKV, DH), lambda h: (h, 0, 0)),
        ],
        out_specs=pl.BlockSpec((1, 32, DH), lambda h: (h, 0, 0)),
        out_shape=jax.ShapeDtypeStruct((HQ_LOC, 32, DH), jnp.float32),
    )(Q, Kg, Vg)


WIN = 1024


def _attention_band(Q, Kw, Vw):

    def body(q_ref, k_ref, v_ref, o_ref):
        qb = pl.program_id(1)
        start = jnp.maximum(qb - 1, 0) * QB
        q = q_ref[0]
        k = k_ref[0, 0]
        v = v_ref[0, 0]
        s = jax.lax.dot_general(
            q, k, (((1,), (1,)), ((), ())),
            preferred_element_type=jnp.float32,
        ) * SCALE
        qi = qb * QB + lax.broadcasted_iota(jnp.int32, (QB, WIN), 0)
        c = lax.broadcasted_iota(jnp.int32, (QB, WIN), 1)
        in_win = c >= QB
        ki = jnp.where(in_win, start + c - QB, c)
        band = jnp.abs(qi - ki) <= 128
        glob = ki < 32
        mask = (in_win & (band | glob)) | ((~in_win) & (start > 0) & glob)
        s = jnp.where(mask, s, -1e9)
        m = jnp.max(s, axis=-1, keepdims=True)
        w = jnp.exp(s - m)
        w = w / jnp.sum(w, axis=-1, keepdims=True)
        o_ref[0] = jax.lax.dot_general(
            w.astype(v.dtype), v, (((1,), (0,)), ((), ())),
            preferred_element_type=jnp.float32,
        )

    return pl.pallas_call(
        body,
        grid=(HQ_LOC, SQ // QB),
        in_specs=[
            pl.BlockSpec((1, QB, DH), lambda h, qb: (h, qb, 0)),
            pl.BlockSpec((1, 1, WIN, DH), lambda h, qb: (qb, h, 0, 0)),
            pl.BlockSpec((1, 1, WIN, DH), lambda h, qb: (qb, h, 0, 0)),
        ],
        out_specs=pl.BlockSpec((1, QB, DH), lambda h, qb: (h, qb, 0)),
        out_shape=jax.ShapeDtypeStruct((HQ_LOC, SQ, DH), jnp.float32),
    )(Q, Kw, Vw)


def _all_gather(partial):

    def body(p_ref, g_ref, send, recv, loc_sem):
        my = lax.axis_index("i")
        barrier = pltpu.get_barrier_semaphore()
        for d in range(1, N_DEV):
            pl.semaphore_signal(
                barrier, inc=1,
                device_id=(lax.rem(my + d, N_DEV),),
                device_id_type=pl.DeviceIdType.MESH,
            )
        pl.semaphore_wait(barrier, N_DEV - 1)
        loc = pltpu.make_async_copy(p_ref, g_ref.at[my], loc_sem)
        loc.start()
        rdmas = []
        for d in range(1, N_DEV):
            t = lax.rem(my + d, N_DEV)
            r = pltpu.make_async_remote_copy(
                src_ref=p_ref,
                dst_ref=g_ref.at[my],
                send_sem=send.at[d - 1],
                recv_sem=recv.at[d - 1],
                device_id=(t,),
                device_id_type=pl.DeviceIdType.MESH,
            )
            r.start()
            rdmas.append(r)
        loc.wait()
        for r in rdmas:
            r.wait()

    return pl.pallas_call(
        body,
        out_shape=jax.ShapeDtypeStruct((N_DEV, SQ, HQ_LOC * DH), partial.dtype),
        in_specs=[pl.BlockSpec(memory_space=pl.ANY)],
        out_specs=pl.BlockSpec(memory_space=pl.ANY),
        scratch_shapes=[
            pltpu.SemaphoreType.DMA((N_DEV - 1,)),
            pltpu.SemaphoreType.DMA((N_DEV - 1,)),
            pltpu.SemaphoreType.DMA(()),
        ],
        compiler_params=pltpu.CompilerParams(collective_id=1),
    )(partial)


def kernel(x, Wq, K_ext, V_ext, Wo):
    bf16 = jnp.bfloat16
    Q = (x[0] @ Wq).reshape(SQ, HQ_LOC, DH).transpose(1, 0, 2).astype(bf16)

    Kt = K_ext[0].astype(bf16).reshape(SKV_LOC, N_DEV, HQ_LOC, DH).transpose(1, 2, 0, 3)
    Vt = V_ext[0].astype(bf16).reshape(SKV_LOC, N_DEV, HQ_LOC, DH).transpose(1, 2, 0, 3)

    Kg, Vg = _all_to_all_kv(Kt, Vt)

    def windows(A):
        return jnp.stack(
            [
                jnp.concatenate(
                    [A[:, :QB], A[:, max(qb - 1, 0) * QB:(max(qb - 1, 0) + 3) * QB]],
                    axis=1,
                )
                for qb in range(SQ // QB)
            ]
        )

    Kw, Vw = windows(Kg), windows(Vg)
    ctx = _attention_band(Q, Kw, Vw)
    ctx_glob = _attention_global_rows(Q, Kg, Vg)
    ctx = jnp.concatenate([ctx_glob, ctx[:, 32:]], axis=1)

    partial = ctx.transpose(1, 0, 2).reshape(SQ, HQ_LOC * DH) @ Wo
    gathered = _all_gather(partial)
    return jnp.sum(gathered, axis=0)[None]


# device time: 485426 ns/iter; 2.5294x vs baseline; 1.2191x over previous
import jax
import jax.numpy as jnp
from jax import lax
from jax.experimental import pallas as pl
from jax.experimental.pallas import tpu as pltpu

N_DEV = 4
HQ_LOC = 8
DH = 128
SQ = 2048
SKV_LOC = 2048
SKV = N_DEV * SKV_LOC
SCALE = 0.08838834764831843
QB = 256


def _all_to_all_kv(Kt, Vt):

    def body(kt_ref, vt_ref, kg_ref, vg_ref,
             send_k, recv_k, send_v, recv_v, loc_sems):
        my = lax.axis_index("i")

        barrier = pltpu.get_barrier_semaphore()
        for d in range(1, N_DEV):
            pl.semaphore_signal(
                barrier, inc=1,
                device_id=(lax.rem(my + d, N_DEV),),
                device_id_type=pl.DeviceIdType.MESH,
            )
        pl.semaphore_wait(barrier, N_DEV - 1)

        kloc = pltpu.make_async_copy(
            kt_ref.at[my], kg_ref.at[:, pl.ds(my * SKV_LOC, SKV_LOC), :], loc_sems.at[0]
        )
        kloc.start()
        vloc = pltpu.make_async_copy(
            vt_ref.at[my], vg_ref.at[:, pl.ds(my * SKV_LOC, SKV_LOC), :], loc_sems.at[1]
        )
        vloc.start()

        rdmas = []
        for d in range(1, N_DEV):
            t = lax.rem(my + d, N_DEV)
            rk = pltpu.make_async_remote_copy(
                src_ref=kt_ref.at[t],
                dst_ref=kg_ref.at[:, pl.ds(my * SKV_LOC, SKV_LOC), :],
                send_sem=send_k.at[d - 1],
                recv_sem=recv_k.at[d - 1],
                device_id=(t,),
                device_id_type=pl.DeviceIdType.MESH,
            )
            rk.start()
            rv = pltpu.make_async_remote_copy(
                src_ref=vt_ref.at[t],
                dst_ref=vg_ref.at[:, pl.ds(my * SKV_LOC, SKV_LOC), :],
                send_sem=send_v.at[d - 1],
                recv_sem=recv_v.at[d - 1],
                device_id=(t,),
                device_id_type=pl.DeviceIdType.MESH,
            )
            rv.start()
            rdmas.append((rk, rv))

        kloc.wait()
        vloc.wait()
        for rk, rv in rdmas:
            rk.wait()
            rv.wait()

    return pl.pallas_call(
        body,
        out_shape=[
            jax.ShapeDtypeStruct((HQ_LOC, SKV, DH), Kt.dtype),
            jax.ShapeDtypeStruct((HQ_LOC, SKV, DH), Vt.dtype),
        ],
        in_specs=[
            pl.BlockSpec(memory_space=pl.ANY),
            pl.BlockSpec(memory_space=pl.ANY),
        ],
        out_specs=[
            pl.BlockSpec(memory_space=pl.ANY),
            pl.BlockSpec(memory_space=pl.ANY),
        ],
        scratch_shapes=[
            pltpu.SemaphoreType.DMA((N_DEV - 1,)),
            pltpu.SemaphoreType.DMA((N_DEV - 1,)),
            pltpu.SemaphoreType.DMA((N_DEV - 1,)),
            pltpu.SemaphoreType.DMA((N_DEV - 1,)),
            pltpu.SemaphoreType.DMA((2,)),
        ],
        compiler_params=pltpu.CompilerParams(collective_id=0),
    )(Kt, Vt)


def _attention_global_rows(Q, Kg, Vg):

    def body(q_ref, k_ref, v_ref, o_ref):
        q = q_ref[0, :32, :]
        k = k_ref[0]
        v = v_ref[0]
        s = jax.lax.dot_general(
            q, k, (((1,), (1,)), ((), ())),
            preferred_element_type=jnp.float32,
        ) * SCALE
        m = jnp.max(s, axis=-1, keepdims=True)
        w = jnp.exp(s - m)
        w = w / jnp.sum(w, axis=-1, keepdims=True)
        o_ref[0] = jax.lax.dot_general(
            w.astype(v.dtype), v, (((1,), (0,)), ((), ())),
            preferred_element_type=jnp.float32,
        )

    return pl.pallas_call(
        body,
        grid=(HQ_LOC,),
        in_specs=[
            pl.BlockSpec((1, QB, DH), lambda h: (h, 0, 0)),
            pl.BlockSpec((1, SKV, DH), lambda h: (h, 0, 0)),
            pl.BlockSpec((1, SKV, DH), lambda h: (h, 0, 0)),
        ],
        out_specs=pl.BlockSpec((1, 32, DH), lambda h: (h, 0, 0)),
        out_shape=jax.ShapeDtypeStruct((HQ_LOC, 32, DH), jnp.float32),
    )(Q, Kg, Vg)


WIN = 1024


def _attention_band(Q, Kw, Vw):

    def body(q_ref, k_ref, v_ref, o_ref):
        qb = pl.program_id(1)
        start = jnp.maximum(qb - 1, 0) * QB
        q = q_ref[0]
        k = k_ref[0, 0]
        v = v_ref[0, 0]
        s = jax.lax.dot_general(
            q, k, (((1,), (1,)), ((), ())),
            preferred_element_type=jnp.float32,
        ) * SCALE
        qi = qb * QB + lax.broadcasted_iota(jnp.int32, (QB, WIN), 0)
        c = lax.broadcasted_iota(jnp.int32, (QB, WIN), 1)
        in_win = c >= QB
        ki = jnp.where(in_win, start + c - QB, c)
        band = jnp.abs(qi - ki) <= 128
        glob = ki < 32
        mask = (in_win & (band | glob)) | ((~in_win) & (start > 0) & glob)
        s = jnp.where(mask, s, -1e9)
        m = jnp.max(s, axis=-1, keepdims=True)
        w = jnp.exp(s - m)
        w = w / jnp.sum(w, axis=-1, keepdims=True)
        o_ref[0] = jax.lax.dot_general(
            w.astype(v.dtype), v, (((1,), (0,)), ((), ())),
            preferred_element_type=jnp.float32,
        )

    return pl.pallas_call(
        body,
        grid=(HQ_LOC, SQ // QB),
        in_specs=[
            pl.BlockSpec((1, QB, DH), lambda h, qb: (h, qb, 0)),
            pl.BlockSpec((1, 1, WIN, DH), lambda h, qb: (qb, h, 0, 0)),
            pl.BlockSpec((1, 1, WIN, DH), lambda h, qb: (qb, h, 0, 0)),
        ],
        out_specs=pl.BlockSpec((1, QB, DH), lambda h, qb: (h, qb, 0)),
        out_shape=jax.ShapeDtypeStruct((HQ_LOC, SQ, DH), jnp.float32),
    )(Q, Kw, Vw)


def _all_gather(partial):

    def body(p_ref, g_ref, send, recv, loc_sem):
        my = lax.axis_index("i")
        barrier = pltpu.get_barrier_semaphore()
        for d in range(1, N_DEV):
            pl.semaphore_signal(
                barrier, inc=1,
                device_id=(lax.rem(my + d, N_DEV),),
                device_id_type=pl.DeviceIdType.MESH,
            )
        pl.semaphore_wait(barrier, N_DEV - 1)
        loc = pltpu.make_async_copy(p_ref, g_ref.at[my], loc_sem)
        loc.start()
        rdmas = []
        for d in range(1, N_DEV):
            t = lax.rem(my + d, N_DEV)
            r = pltpu.make_async_remote_copy(
                src_ref=p_ref,
                dst_ref=g_ref.at[my],
                send_sem=send.at[d - 1],
                recv_sem=recv.at[d - 1],
                device_id=(t,),
                device_id_type=pl.DeviceIdType.MESH,
            )
            r.start()
            rdmas.append(r)
        loc.wait()
        for r in rdmas:
            r.wait()

    return pl.pallas_call(
        body,
        out_shape=jax.ShapeDtypeStruct((N_DEV, SQ, HQ_LOC * DH), partial.dtype),
        in_specs=[pl.BlockSpec(memory_space=pl.ANY)],
        out_specs=pl.BlockSpec(memory_space=pl.ANY),
        scratch_shapes=[
            pltpu.SemaphoreType.DMA((N_DEV - 1,)),
            pltpu.SemaphoreType.DMA((N_DEV - 1,)),
            pltpu.SemaphoreType.DMA(()),
        ],
        compiler_params=pltpu.CompilerParams(collective_id=1),
    )(partial)


def kernel(x, Wq, K_ext, V_ext, Wo):
    bf16 = jnp.bfloat16
    Q = (x[0].astype(bf16) @ Wq.astype(bf16)).reshape(SQ, HQ_LOC, DH).transpose(1, 0, 2)

    Kt = K_ext[0].astype(bf16).reshape(SKV_LOC, N_DEV, HQ_LOC, DH).transpose(1, 2, 0, 3)
    Vt = V_ext[0].astype(bf16).reshape(SKV_LOC, N_DEV, HQ_LOC, DH).transpose(1, 2, 0, 3)

    Kg, Vg = _all_to_all_kv(Kt, Vt)

    def windows(A):
        return jnp.stack(
            [
                jnp.concatenate(
                    [A[:, :QB], A[:, max(qb - 1, 0) * QB:(max(qb - 1, 0) + 3) * QB]],
                    axis=1,
                )
                for qb in range(SQ // QB)
            ]
        )

    Kw, Vw = windows(Kg), windows(Vg)
    ctx = _attention_band(Q, Kw, Vw)
    ctx_glob = _attention_global_rows(Q, Kg, Vg)
    ctx = jnp.concatenate([ctx_glob, ctx[:, 32:]], axis=1)

    partial = (
        ctx.transpose(1, 0, 2).reshape(SQ, HQ_LOC * DH).astype(bf16)
        @ Wo.astype(bf16)
    )
    gathered = _all_gather(partial)
    return jnp.sum(gathered.astype(jnp.float32), axis=0)[None]


# device time: 405419 ns/iter; 3.0286x vs baseline; 1.1973x over previous
import jax
import jax.numpy as jnp
from jax import lax
from jax.experimental import pallas as pl
from jax.experimental.pallas import tpu as pltpu

N_DEV = 4
HQ_LOC = 8
DH = 128
SQ = 2048
SKV_LOC = 2048
SKV = N_DEV * SKV_LOC
SCALE = 0.08838834764831843
QB = 256


def _all_to_all_kv(Kt, Vt):

    def body(kt_ref, vt_ref, kg_ref, vg_ref,
             send_k, recv_k, send_v, recv_v, loc_sems):
        my = lax.axis_index("i")

        barrier = pltpu.get_barrier_semaphore()
        for d in range(1, N_DEV):
            pl.semaphore_signal(
                barrier, inc=1,
                device_id=(lax.rem(my + d, N_DEV),),
                device_id_type=pl.DeviceIdType.MESH,
            )
        pl.semaphore_wait(barrier, N_DEV - 1)

        kloc = pltpu.make_async_copy(
            kt_ref.at[my], kg_ref.at[:, pl.ds(my * SKV_LOC, SKV_LOC), :], loc_sems.at[0]
        )
        kloc.start()
        vloc = pltpu.make_async_copy(
            vt_ref.at[my], vg_ref.at[:, pl.ds(my * SKV_LOC, SKV_LOC), :], loc_sems.at[1]
        )
        vloc.start()

        rdmas = []
        for d in range(1, N_DEV):
            t = lax.rem(my + d, N_DEV)
            rk = pltpu.make_async_remote_copy(
                src_ref=kt_ref.at[t],
                dst_ref=kg_ref.at[:, pl.ds(my * SKV_LOC, SKV_LOC), :],
                send_sem=send_k.at[d - 1],
                recv_sem=recv_k.at[d - 1],
                device_id=(t,),
                device_id_type=pl.DeviceIdType.MESH,
            )
            rk.start()
            rv = pltpu.make_async_remote_copy(
                src_ref=vt_ref.at[t],
                dst_ref=vg_ref.at[:, pl.ds(my * SKV_LOC, SKV_LOC), :],
                send_sem=send_v.at[d - 1],
                recv_sem=recv_v.at[d - 1],
                device_id=(t,),
                device_id_type=pl.DeviceIdType.MESH,
            )
            rv.start()
            rdmas.append((rk, rv))

        kloc.wait()
        vloc.wait()
        for rk, rv in rdmas:
            rk.wait()
            rv.wait()

    return pl.pallas_call(
        body,
        out_shape=[
            jax.ShapeDtypeStruct((HQ_LOC, SKV, DH), Kt.dtype),
            jax.ShapeDtypeStruct((HQ_LOC, SKV, DH), Vt.dtype),
        ],
        in_specs=[
            pl.BlockSpec(memory_space=pl.ANY),
            pl.BlockSpec(memory_space=pl.ANY),
        ],
        out_specs=[
            pl.BlockSpec(memory_space=pl.ANY),
            pl.BlockSpec(memory_space=pl.ANY),
        ],
        scratch_shapes=[
            pltpu.SemaphoreType.DMA((N_DEV - 1,)),
            pltpu.SemaphoreType.DMA((N_DEV - 1,)),
            pltpu.SemaphoreType.DMA((N_DEV - 1,)),
            pltpu.SemaphoreType.DMA((N_DEV - 1,)),
            pltpu.SemaphoreType.DMA((2,)),
        ],
        compiler_params=pltpu.CompilerParams(collective_id=0),
    )(Kt, Vt)


def _attention_global_rows(Q, Kg, Vg):

    def body(q_ref, k_ref, v_ref, o_ref):
        q = q_ref[0, :32, :]
        k = k_ref[0]
        v = v_ref[0]
        s = jax.lax.dot_general(
            q, k, (((1,), (1,)), ((), ())),
            preferred_element_type=jnp.float32,
        ) * SCALE
        m = jnp.max(s, axis=-1, keepdims=True)
        w = jnp.exp(s - m)
        w = w / jnp.sum(w, axis=-1, keepdims=True)
        o_ref[0] = jax.lax.dot_general(
            w.astype(v.dtype), v, (((1,), (0,)), ((), ())),
            preferred_element_type=jnp.float32,
        )

    return pl.pallas_call(
        body,
        grid=(HQ_LOC,),
        in_specs=[
            pl.BlockSpec((1, QB, DH), lambda h: (h, 0, 0)),
            pl.BlockSpec((1, SKV, DH), lambda h: (h, 0, 0)),
            pl.BlockSpec((1, SKV, DH), lambda h: (h, 0, 0)),
        ],
        out_specs=pl.BlockSpec((1, 32, DH), lambda h: (h, 0, 0)),
        out_shape=jax.ShapeDtypeStruct((HQ_LOC, 32, DH), jnp.float32),
    )(Q, Kg, Vg)


GW = 32
WIN = 3 * QB


def _attention_band(Q, Kg, Vg):

    def body(q_ref, k0_ref, ka_ref, kb_ref, kc_ref,
             v0_ref, va_ref, vb_ref, vc_ref, o_ref):
        qb = pl.program_id(1)
        start = jnp.maximum(qb - 1, 0) * QB
        q = q_ref[0]
        k = jnp.concatenate(
            [k0_ref[0], ka_ref[0], kb_ref[0], kc_ref[0]], axis=0
        )
        v = jnp.concatenate(
            [v0_ref[0], va_ref[0], vb_ref[0], vc_ref[0]], axis=0
        )
        s = jax.lax.dot_general(
            q, k, (((1,), (1,)), ((), ())),
            preferred_element_type=jnp.float32,
        ) * SCALE
        qi = qb * QB + lax.broadcasted_iota(jnp.int32, (QB, GW + WIN), 0)
        c = lax.broadcasted_iota(jnp.int32, (QB, GW + WIN), 1)
        in_win = c >= GW
        ki = jnp.where(in_win, start + c - GW, c)
        band = jnp.abs(qi - ki) <= 128
        glob = ki < 32
        mask = (in_win & (band | glob)) | ((~in_win) & (start > 0))
        s = jnp.where(mask, s, -1e9)
        m = jnp.max(s, axis=-1, keepdims=True)
        w = jnp.exp(s - m)
        w = w / jnp.sum(w, axis=-1, keepdims=True)
        o_ref[0] = jax.lax.dot_general(
            w.astype(v.dtype), v, (((1,), (0,)), ((), ())),
            preferred_element_type=jnp.float32,
        )

    win_spec = lambda off: pl.BlockSpec(
        (1, QB, DH), lambda h, qb: (h, jnp.maximum(qb - 1, 0) + off, 0)
    )
    glob_spec = pl.BlockSpec((1, GW, DH), lambda h, qb: (h, 0, 0))
    return pl.pallas_call(
        body,
        grid=(HQ_LOC, SQ // QB),
        in_specs=[
            pl.BlockSpec((1, QB, DH), lambda h, qb: (h, qb, 0)),
            glob_spec, win_spec(0), win_spec(1), win_spec(2),
            glob_spec, win_spec(0), win_spec(1), win_spec(2),
        ],
        out_specs=pl.BlockSpec((1, QB, DH), lambda h, qb: (h, qb, 0)),
        out_shape=jax.ShapeDtypeStruct((HQ_LOC, SQ, DH), jnp.float32),
    )(Q, Kg, Kg, Kg, Kg, Vg, Vg, Vg, Vg)


def _reduce_scatter(partial):
    rows = SQ // N_DEV

    def body(p_ref, g_ref, send, recv, loc_sem):
        my = lax.axis_index("i")
        barrier = pltpu.get_barrier_semaphore()
        for d in range(1, N_DEV):
            pl.semaphore_signal(
                barrier, inc=1,
                device_id=(lax.rem(my + d, N_DEV),),
                device_id_type=pl.DeviceIdType.MESH,
            )
        pl.semaphore_wait(barrier, N_DEV - 1)
        loc = pltpu.make_async_copy(
            p_ref.at[pl.ds(my * rows, rows)], g_ref.at[my], loc_sem
        )
        loc.start()
        rdmas = []
        for d in range(1, N_DEV):
            t = lax.rem(my + d, N_DEV)
            r = pltpu.make_async_remote_copy(
                src_ref=p_ref.at[pl.ds(t * rows, rows)],
                dst_ref=g_ref.at[my],
                send_sem=send.at[d - 1],
                recv_sem=recv.at[d - 1],
                device_id=(t,),
                device_id_type=pl.DeviceIdType.MESH,
            )
            r.start()
            rdmas.append(r)
        loc.wait()
        for r in rdmas:
            r.wait()

    return pl.pallas_call(
        body,
        out_shape=jax.ShapeDtypeStruct((N_DEV, rows, HQ_LOC * DH), partial.dtype),
        in_specs=[pl.BlockSpec(memory_space=pl.ANY)],
        out_specs=pl.BlockSpec(memory_space=pl.ANY),
        scratch_shapes=[
            pltpu.SemaphoreType.DMA((N_DEV - 1,)),
            pltpu.SemaphoreType.DMA((N_DEV - 1,)),
            pltpu.SemaphoreType.DMA(()),
        ],
        compiler_params=pltpu.CompilerParams(collective_id=1),
    )(partial)


def _all_gather(q_sum):
    rows = SQ // N_DEV

    def body(p_ref, g_ref, send, recv, loc_sem):
        my = lax.axis_index("i")
        barrier = pltpu.get_barrier_semaphore()
        for d in range(1, N_DEV):
            pl.semaphore_signal(
                barrier, inc=1,
                device_id=(lax.rem(my + d, N_DEV),),
                device_id_type=pl.DeviceIdType.MESH,
            )
        pl.semaphore_wait(barrier, N_DEV - 1)
        loc = pltpu.make_async_copy(p_ref, g_ref.at[my], loc_sem)
        loc.start()
        rdmas = []
        for d in range(1, N_DEV):
            t = lax.rem(my + d, N_DEV)
            r = pltpu.make_async_remote_copy(
                src_ref=p_ref,
                dst_ref=g_ref.at[my],
                send_sem=send.at[d - 1],
                recv_sem=recv.at[d - 1],
                device_id=(t,),
                device_id_type=pl.DeviceIdType.MESH,
            )
            r.start()
            rdmas.append(r)
        loc.wait()
        for r in rdmas:
            r.wait()

    return pl.pallas_call(
        body,
        out_shape=jax.ShapeDtypeStruct((N_DEV, rows, HQ_LOC * DH), q_sum.dtype),
        in_specs=[pl.BlockSpec(memory_space=pl.ANY)],
        out_specs=pl.BlockSpec(memory_space=pl.ANY),
        scratch_shapes=[
            pltpu.SemaphoreType.DMA((N_DEV - 1,)),
            pltpu.SemaphoreType.DMA((N_DEV - 1,)),
            pltpu.SemaphoreType.DMA(()),
        ],
        compiler_params=pltpu.CompilerParams(collective_id=2),
    )(q_sum)


def kernel(x, Wq, K_ext, V_ext, Wo):
    bf16 = jnp.bfloat16
    Q = (x[0].astype(bf16) @ Wq.astype(bf16)).reshape(SQ, HQ_LOC, DH).transpose(1, 0, 2)

    Kt = K_ext[0].astype(bf16).reshape(SKV_LOC, N_DEV, HQ_LOC, DH).transpose(1, 2, 0, 3)
    Vt = V_ext[0].astype(bf16).reshape(SKV_LOC, N_DEV, HQ_LOC, DH).transpose(1, 2, 0, 3)

    Kg, Vg = _all_to_all_kv(Kt, Vt)

    ctx = _attention_band(Q, Kg, Vg)
    ctx_glob = _attention_global_rows(Q, Kg, Vg)
    ctx = jnp.concatenate([ctx_glob, ctx[:, 32:]], axis=1)

    partial = (
        ctx.transpose(1, 0, 2).reshape(SQ, HQ_LOC * DH).astype(bf16)
        @ Wo.astype(bf16)
    )
    quarters = _reduce_scatter(partial)
    q_sum = jnp.sum(quarters.astype(jnp.float32), axis=0).astype(bf16)
    gathered = _all_gather(q_sum)
    return gathered.reshape(SQ, HQ_LOC * DH).astype(jnp.float32)[None]
